# baseline (speedup 1.0000x reference)
import functools

import jax
import jax.numpy as jnp
from jax.experimental import pallas as pl
from jax.experimental.pallas import tpu as pltpu

_TM = 1024
_TW = 1024


def _ff_kernel(x_ref, w1_ref, b1_ref, w2_ref, b2_ref, o_ref, *, n_w):
    w = pl.program_id(1)

    h = jnp.dot(x_ref[...], w1_ref[...], preferred_element_type=jnp.float32)
    h = jnp.maximum(h + b1_ref[...], 0.0).astype(jnp.bfloat16)
    contrib = jnp.dot(h, w2_ref[...], preferred_element_type=jnp.float32)

    @pl.when(w == 0)
    def _init():
        o_ref[...] = contrib + b2_ref[...]

    @pl.when(w != 0)
    def _acc():
        o_ref[...] += contrib


def kernel(x, gate, W1, b1, W2, b2):
    batch, cutoff, dmodel = x.shape
    n_tokens = batch * cutoff
    width = W1.shape[1]

    x2 = x.reshape(n_tokens, dmodel).astype(jnp.bfloat16)
    w1 = W1.astype(jnp.bfloat16)
    w2 = W2.astype(jnp.bfloat16)
    b1f = b1.astype(jnp.float32).reshape(1, width)
    b2f = b2.astype(jnp.float32).reshape(1, dmodel)

    n_m = n_tokens // _TM
    n_w = width // _TW

    out = pl.pallas_call(
        functools.partial(_ff_kernel, n_w=n_w),
        grid=(n_m, n_w),
        in_specs=[
            pl.BlockSpec((_TM, dmodel), lambda m, w: (m, 0)),
            pl.BlockSpec((dmodel, _TW), lambda m, w: (0, w)),
            pl.BlockSpec((1, _TW), lambda m, w: (0, w)),
            pl.BlockSpec((_TW, dmodel), lambda m, w: (w, 0)),
            pl.BlockSpec((1, dmodel), lambda m, w: (0, 0)),
        ],
        out_specs=pl.BlockSpec((_TM, dmodel), lambda m, w: (m, 0)),
        out_shape=jax.ShapeDtypeStruct((n_tokens, dmodel), jnp.float32),
        compiler_params=pltpu.CompilerParams(
            dimension_semantics=("parallel", "arbitrary"),
        ),
    )(x2, w1, b1f, w2, b2f)

    return out.reshape(batch, cutoff, dmodel)


# speedup vs baseline: 1.0263x; 1.0263x over previous
import functools

import jax
import jax.numpy as jnp
from jax.experimental import pallas as pl
from jax.experimental.pallas import tpu as pltpu

_TM = 512
_TW = 2048
_SUB = 512


def _ff_kernel(x_ref, w1_ref, b1_ref, w2_ref, b2_ref, o_ref, *, n_w):
    w = pl.program_id(1)
    x = x_ref[...]

    parts = []
    for j in range(_TW // _SUB):
        sl = slice(j * _SUB, (j + 1) * _SUB)
        h = jnp.dot(x, w1_ref[:, sl], preferred_element_type=jnp.float32)
        h = jnp.maximum(h + b1_ref[:, sl], 0.0).astype(jnp.bfloat16)
        parts.append(
            jnp.dot(h, w2_ref[sl, :], preferred_element_type=jnp.float32)
        )
    while len(parts) > 1:
        parts = [a + b for a, b in zip(parts[::2], parts[1::2])]
    contrib = parts[0]

    @pl.when(w == 0)
    def _init():
        o_ref[...] = contrib + b2_ref[...]

    @pl.when(w != 0)
    def _acc():
        o_ref[...] += contrib


def kernel(x, gate, W1, b1, W2, b2):
    batch, cutoff, dmodel = x.shape
    n_tokens = batch * cutoff
    width = W1.shape[1]

    x2 = x.reshape(n_tokens, dmodel).astype(jnp.bfloat16)
    w1 = W1.astype(jnp.bfloat16)
    w2 = W2.astype(jnp.bfloat16)
    b1f = b1.astype(jnp.float32).reshape(1, width)
    b2f = b2.astype(jnp.float32).reshape(1, dmodel)

    n_m = n_tokens // _TM
    n_w = width // _TW

    out = pl.pallas_call(
        functools.partial(_ff_kernel, n_w=n_w),
        grid=(n_m, n_w),
        in_specs=[
            pl.BlockSpec((_TM, dmodel), lambda m, w: (m, 0)),
            pl.BlockSpec((dmodel, _TW), lambda m, w: (0, w)),
            pl.BlockSpec((1, _TW), lambda m, w: (0, w)),
            pl.BlockSpec((_TW, dmodel), lambda m, w: (w, 0)),
            pl.BlockSpec((1, dmodel), lambda m, w: (0, 0)),
        ],
        out_specs=pl.BlockSpec((_TM, dmodel), lambda m, w: (m, 0)),
        out_shape=jax.ShapeDtypeStruct((n_tokens, dmodel), jnp.float32),
        compiler_params=pltpu.CompilerParams(
            dimension_semantics=("parallel", "arbitrary"),
        ),
    )(x2, w1, b1f, w2, b2f)

    return out.reshape(batch, cutoff, dmodel)
